# Initial kernel scaffold; baseline (speedup 1.0000x reference)
import jax
import jax.numpy as jnp
from jax.experimental import pallas as pl


def kernel(x, cat_1, cat_2, cat_3, occupation, emb1, emb2, emb3, emb_occ, W1, b1, W2, b2, W3, b3):
    raise NotImplementedError("write your pallas kernel here")


# speedup vs baseline: 7.3705x; 7.3705x over previous
import functools

import jax
import jax.numpy as jnp
from jax.experimental import pallas as pl
from jax.experimental.pallas import tpu as pltpu

B = 16384
BB = 2048
NB = B // BB

_SLOTS = ((13, 20), (45, 18), (77, 16), (109, 21))


def _fwd_kernel(x_ref, idx_ref, e1_ref, e2_ref, e3_ref, e4_ref,
                w1_ref, b1_ref, w2_ref, b2_ref, w3_ref, b3_ref, o_ref):
    f32 = jnp.float32
    acc = jnp.dot(x_ref[...], w1_ref[0:13, :], preferred_element_type=f32)
    acc = acc + b1_ref[...]
    embs = (e1_ref, e2_ref, e3_ref, e4_ref)
    for s, (w_lo, k) in enumerate(_SLOTS):
        t_s = jnp.dot(embs[s][...], w1_ref[w_lo:w_lo + 32, :],
                      preferred_element_type=f32)
        ids = idx_ref[0, s, :]
        onehot = (ids[:, None] ==
                  jax.lax.broadcasted_iota(jnp.int32, (BB, k), 1)).astype(f32)
        acc = acc + jnp.dot(onehot, t_s, preferred_element_type=f32)
    h1 = jnp.maximum(acc, 0.0)
    h2 = jnp.maximum(
        jnp.dot(h1, w2_ref[...], preferred_element_type=f32) + b2_ref[...], 0.0)
    o_ref[...] = jnp.dot(h2, w3_ref[...], preferred_element_type=f32) + b3_ref[...]


@functools.partial(jax.jit, static_argnums=())
def kernel(x, cat_1, cat_2, cat_3, occupation,
           emb1, emb2, emb3, emb_occ, W1, b1, W2, b2, W3, b3):
    idx = jnp.stack([cat_1, cat_2, cat_3, occupation]).astype(jnp.int32)
    idx = idx.reshape(4, NB, BB).transpose(1, 0, 2)
    b1r = b1.reshape(1, 128)
    b2r = b2.reshape(1, 64)
    b3r = b3.reshape(1, 1)

    grid = (NB,)
    whole = lambda *_: tuple(0 for _ in range(2))
    out = pl.pallas_call(
        _fwd_kernel,
        grid=grid,
        in_specs=[
            pl.BlockSpec((BB, 13), lambda i: (i, 0)),
            pl.BlockSpec((1, 4, BB), lambda i: (i, 0, 0)),
            pl.BlockSpec((20, 32), lambda i: (0, 0)),
            pl.BlockSpec((18, 32), lambda i: (0, 0)),
            pl.BlockSpec((16, 32), lambda i: (0, 0)),
            pl.BlockSpec((21, 32), lambda i: (0, 0)),
            pl.BlockSpec((141, 128), lambda i: (0, 0)),
            pl.BlockSpec((1, 128), lambda i: (0, 0)),
            pl.BlockSpec((128, 64), lambda i: (0, 0)),
            pl.BlockSpec((1, 64), lambda i: (0, 0)),
            pl.BlockSpec((64, 1), lambda i: (0, 0)),
            pl.BlockSpec((1, 1), lambda i: (0, 0)),
        ],
        out_specs=pl.BlockSpec((BB, 1), lambda i: (i, 0)),
        out_shape=jax.ShapeDtypeStruct((B, 1), jnp.float32),
        compiler_params=pltpu.CompilerParams(
            dimension_semantics=("arbitrary",)),
    )(x, idx, emb1, emb2, emb3, emb_occ, W1, b1r, W2, b2r, W3, b3r)
    return out


# speedup vs baseline: 7.6073x; 1.0321x over previous
import functools

import jax
import jax.numpy as jnp
from jax.experimental import pallas as pl
from jax.experimental.pallas import tpu as pltpu

B = 16384
BB = 4096
NB = B // BB

_SLOTS = ((13, 20), (45, 18), (77, 16), (109, 21))


def _fwd_kernel(x_ref, idx_ref, e1_ref, e2_ref, e3_ref, e4_ref,
                w1_ref, b1_ref, w2_ref, b2_ref, w3_ref, b3_ref, o_ref):
    f32 = jnp.float32
    acc = jnp.dot(x_ref[...], w1_ref[0:13, :], preferred_element_type=f32)
    acc = acc + b1_ref[...]
    embs = (e1_ref, e2_ref, e3_ref, e4_ref)
    for s, (w_lo, k) in enumerate(_SLOTS):
        t_s = jnp.dot(embs[s][...], w1_ref[w_lo:w_lo + 32, :],
                      preferred_element_type=f32)
        ids = idx_ref[0, s, :]
        onehot = (ids[:, None] ==
                  jax.lax.broadcasted_iota(jnp.int32, (BB, k), 1)).astype(f32)
        acc = acc + jnp.dot(onehot, t_s, preferred_element_type=f32)
    h1 = jnp.maximum(acc, 0.0)
    h2 = jnp.maximum(
        jnp.dot(h1, w2_ref[...], preferred_element_type=f32) + b2_ref[...], 0.0)
    o_ref[...] = jnp.dot(h2, w3_ref[...], preferred_element_type=f32) + b3_ref[...]


@functools.partial(jax.jit, static_argnums=())
def kernel(x, cat_1, cat_2, cat_3, occupation,
           emb1, emb2, emb3, emb_occ, W1, b1, W2, b2, W3, b3):
    idx = jnp.stack([cat_1, cat_2, cat_3, occupation]).astype(jnp.int32)
    idx = idx.reshape(4, NB, BB).transpose(1, 0, 2)
    b1r = b1.reshape(1, 128)
    b2r = b2.reshape(1, 64)
    b3r = b3.reshape(1, 1)

    grid = (NB,)
    whole = lambda *_: tuple(0 for _ in range(2))
    out = pl.pallas_call(
        _fwd_kernel,
        grid=grid,
        in_specs=[
            pl.BlockSpec((BB, 13), lambda i: (i, 0)),
            pl.BlockSpec((1, 4, BB), lambda i: (i, 0, 0)),
            pl.BlockSpec((20, 32), lambda i: (0, 0)),
            pl.BlockSpec((18, 32), lambda i: (0, 0)),
            pl.BlockSpec((16, 32), lambda i: (0, 0)),
            pl.BlockSpec((21, 32), lambda i: (0, 0)),
            pl.BlockSpec((141, 128), lambda i: (0, 0)),
            pl.BlockSpec((1, 128), lambda i: (0, 0)),
            pl.BlockSpec((128, 64), lambda i: (0, 0)),
            pl.BlockSpec((1, 64), lambda i: (0, 0)),
            pl.BlockSpec((64, 1), lambda i: (0, 0)),
            pl.BlockSpec((1, 1), lambda i: (0, 0)),
        ],
        out_specs=pl.BlockSpec((BB, 1), lambda i: (i, 0)),
        out_shape=jax.ShapeDtypeStruct((B, 1), jnp.float32),
        compiler_params=pltpu.CompilerParams(
            dimension_semantics=("arbitrary",)),
    )(x, idx, emb1, emb2, emb3, emb_occ, W1, b1r, W2, b2r, W3, b3r)
    return out


# speedup vs baseline: 7.6226x; 1.0020x over previous
import functools

import jax
import jax.numpy as jnp
from jax.experimental import pallas as pl
from jax.experimental.pallas import tpu as pltpu

B = 16384
BB = 4096
NB = B // BB

_SLOTS = ((13, 20), (45, 18), (77, 16), (109, 21))


def _fwd_kernel(x_ref, idx_ref, e1_ref, e2_ref, e3_ref, e4_ref,
                w1_ref, b1_ref, w2_ref, b2_ref, w3_ref, b3_ref, o_ref):
    f32 = jnp.float32
    acc = jnp.dot(x_ref[...], w1_ref[0:13, :], preferred_element_type=f32)
    acc = acc + b1_ref[...]
    embs = (e1_ref, e2_ref, e3_ref, e4_ref)
    for s, (w_lo, k) in enumerate(_SLOTS):
        t_s = jnp.dot(embs[s][...], w1_ref[w_lo:w_lo + 32, :],
                      preferred_element_type=f32)
        ids = idx_ref[s, :]
        onehot = (ids[:, None] ==
                  jax.lax.broadcasted_iota(jnp.int32, (BB, k), 1)).astype(f32)
        acc = acc + jnp.dot(onehot, t_s, preferred_element_type=f32)
    h1 = jnp.maximum(acc, 0.0)
    h2 = jnp.maximum(
        jnp.dot(h1, w2_ref[...], preferred_element_type=f32) + b2_ref[...], 0.0)
    o_ref[...] = jnp.dot(h2, w3_ref[...], preferred_element_type=f32) + b3_ref[...]


@functools.partial(jax.jit, static_argnums=())
def kernel(x, cat_1, cat_2, cat_3, occupation,
           emb1, emb2, emb3, emb_occ, W1, b1, W2, b2, W3, b3):
    idx = jnp.stack([cat_1, cat_2, cat_3, occupation]).astype(jnp.int32)
    b1r = b1.reshape(1, 128)
    b2r = b2.reshape(1, 64)
    b3r = b3.reshape(1, 1)

    grid = (NB,)
    whole = lambda *_: tuple(0 for _ in range(2))
    out = pl.pallas_call(
        _fwd_kernel,
        grid=grid,
        in_specs=[
            pl.BlockSpec((BB, 13), lambda i: (i, 0)),
            pl.BlockSpec((4, BB), lambda i: (0, i)),
            pl.BlockSpec((20, 32), lambda i: (0, 0)),
            pl.BlockSpec((18, 32), lambda i: (0, 0)),
            pl.BlockSpec((16, 32), lambda i: (0, 0)),
            pl.BlockSpec((21, 32), lambda i: (0, 0)),
            pl.BlockSpec((141, 128), lambda i: (0, 0)),
            pl.BlockSpec((1, 128), lambda i: (0, 0)),
            pl.BlockSpec((128, 64), lambda i: (0, 0)),
            pl.BlockSpec((1, 64), lambda i: (0, 0)),
            pl.BlockSpec((64, 1), lambda i: (0, 0)),
            pl.BlockSpec((1, 1), lambda i: (0, 0)),
        ],
        out_specs=pl.BlockSpec((BB, 1), lambda i: (i, 0)),
        out_shape=jax.ShapeDtypeStruct((B, 1), jnp.float32),
        compiler_params=pltpu.CompilerParams(
            dimension_semantics=("arbitrary",)),
    )(x, idx, emb1, emb2, emb3, emb_occ, W1, b1r, W2, b2r, W3, b3r)
    return out


# speedup vs baseline: 7.6629x; 1.0053x over previous
import functools

import jax
import jax.numpy as jnp
from jax.experimental import pallas as pl
from jax.experimental.pallas import tpu as pltpu

B = 16384
BB = 4096
NB = B // BB

_SLOTS = ((13, 20, 0), (45, 18, 20), (77, 16, 38), (109, 21, 54))
_TOT = 75


def _fwd_kernel(x_ref, idx_ref, e1_ref, e2_ref, e3_ref, e4_ref,
                w1_ref, b1_ref, w2_ref, b2_ref, w3_ref, b3_ref, o_ref):
    f32 = jnp.float32
    col = jax.lax.broadcasted_iota(jnp.int32, (4, 128), 1)
    row = jax.lax.broadcasted_iota(jnp.int32, (4, 128), 0)
    s_mat = jnp.zeros((4, 128), dtype=f32)
    iota_adj = jnp.full((1, 128), -1, dtype=f32)
    col1 = col[0:1, :]
    for s, (_, k, c_lo) in enumerate(_SLOTS):
        in_range = (col >= c_lo) & (col < c_lo + k)
        s_mat = jnp.where((row == s) & in_range, 1.0, s_mat)
        iota_adj = jnp.where((col1 >= c_lo) & (col1 < c_lo + k),
                             (col1 - c_lo).astype(f32), iota_adj)

    p = jnp.dot(idx_ref[...], s_mat, preferred_element_type=f32)
    m = (p == iota_adj).astype(f32)

    embs = (e1_ref, e2_ref, e3_ref, e4_ref)
    t_parts = [
        jnp.dot(embs[s][...], w1_ref[w_lo:w_lo + 32, :],
                preferred_element_type=f32)
        for s, (w_lo, k, _) in enumerate(_SLOTS)
    ]
    t_parts.append(jnp.zeros((128 - _TOT, 128), dtype=f32))
    t = jnp.concatenate(t_parts, axis=0)

    acc = jnp.dot(x_ref[...], w1_ref[0:13, :], preferred_element_type=f32)
    acc = acc + jnp.dot(m, t, preferred_element_type=f32) + b1_ref[...]
    h1 = jnp.maximum(acc, 0.0)
    h2 = jnp.maximum(
        jnp.dot(h1, w2_ref[...], preferred_element_type=f32) + b2_ref[...], 0.0)
    o_ref[...] = jnp.dot(h2, w3_ref[...], preferred_element_type=f32) + b3_ref[...]


@functools.partial(jax.jit, static_argnums=())
def kernel(x, cat_1, cat_2, cat_3, occupation,
           emb1, emb2, emb3, emb_occ, W1, b1, W2, b2, W3, b3):
    idx = jnp.stack([cat_1, cat_2, cat_3, occupation],
                    axis=1).astype(jnp.float32)
    b1r = b1.reshape(1, 128)
    b2r = b2.reshape(1, 64)
    b3r = b3.reshape(1, 1)

    out = pl.pallas_call(
        _fwd_kernel,
        grid=(NB,),
        in_specs=[
            pl.BlockSpec((BB, 13), lambda i: (i, 0)),
            pl.BlockSpec((BB, 4), lambda i: (i, 0)),
            pl.BlockSpec((20, 32), lambda i: (0, 0)),
            pl.BlockSpec((18, 32), lambda i: (0, 0)),
            pl.BlockSpec((16, 32), lambda i: (0, 0)),
            pl.BlockSpec((21, 32), lambda i: (0, 0)),
            pl.BlockSpec((141, 128), lambda i: (0, 0)),
            pl.BlockSpec((1, 128), lambda i: (0, 0)),
            pl.BlockSpec((128, 64), lambda i: (0, 0)),
            pl.BlockSpec((1, 64), lambda i: (0, 0)),
            pl.BlockSpec((64, 1), lambda i: (0, 0)),
            pl.BlockSpec((1, 1), lambda i: (0, 0)),
        ],
        out_specs=pl.BlockSpec((BB, 1), lambda i: (i, 0)),
        out_shape=jax.ShapeDtypeStruct((B, 1), jnp.float32),
        compiler_params=pltpu.CompilerParams(
            dimension_semantics=("arbitrary",)),
    )(x, idx, emb1, emb2, emb3, emb_occ, W1, b1r, W2, b2r, W3, b3r)
    return out
